# baseline (speedup 1.0000x reference)
import jax
import jax.numpy as jnp
from jax.experimental import pallas as pl
from jax.experimental.pallas import tpu as pltpu


def _gcn_kernel(v_ref, w_ref, adj0_ref, adj1_ref, out_ref, vw_ref):
    @pl.when(pl.program_id(0) == 0)
    def _():
        vw_ref[...] = jax.lax.dot_general(
            v_ref[...], w_ref[...],
            dimension_numbers=(((1,), (1,)), ((), ())),
            preferred_element_type=jnp.float32,
        ).astype(jnp.bfloat16)

    vw = vw_ref[...]
    half = adj0_ref.shape[0]
    out_ref[:half, :] = jnp.maximum(
        jnp.dot(adj0_ref[...].astype(jnp.bfloat16), vw,
                preferred_element_type=jnp.float32),
        0.0,
    )
    out_ref[half:, :] = jnp.maximum(
        jnp.dot(adj1_ref[...].astype(jnp.bfloat16), vw,
                preferred_element_type=jnp.float32),
        0.0,
    )


def kernel(v, adj, W):
    N, d_in = v.shape
    d_out = W.shape[0]

    BM = 200
    out = pl.pallas_call(
        _gcn_kernel,
        grid=(N // (2 * BM),),
        in_specs=[
            pl.BlockSpec((N, d_in), lambda i: (0, 0)),
            pl.BlockSpec((d_out, d_in), lambda i: (0, 0)),
            pl.BlockSpec((BM, N), lambda i: (2 * i, 0)),
            pl.BlockSpec((BM, N), lambda i: (2 * i + 1, 0)),
        ],
        out_specs=pl.BlockSpec((2 * BM, d_out), lambda i: (i, 0)),
        out_shape=jax.ShapeDtypeStruct((N, d_out), jnp.float32),
        scratch_shapes=[pltpu.VMEM((N, d_out), jnp.bfloat16)],
        compiler_params=pltpu.CompilerParams(
            dimension_semantics=("arbitrary",),
        ),
    )(v, W, adj, adj)

    return (out, adj)
